# baseline (device time: 12311 ns/iter reference)
import jax
import jax.numpy as jnp
from jax import lax
from jax.experimental import pallas as pl
from jax.experimental.pallas import tpu as pltpu

V = 4096
T = 512
D = 512
TB = T // 2
C = 8
CH = TB // C


def kernel(ids, E):
    my_x = lax.axis_index("x")
    my_y = lax.axis_index("y")

    ids_blk = lax.dynamic_slice(ids, (my_y * TB,), (TB,))
    local = ids_blk - my_x * V
    local = jnp.where(local < 0, V, local)
    partial = jnp.take(
        E, local, axis=0, mode="fill", fill_value=0.0
    ).astype(jnp.bfloat16)

    def body(partial_ref, out_ref, commx_ref, sendy_ref, commy_ref,
             sx_send, sx_recv, sy_send, sy_recv):
        my_x = lax.axis_index("x")
        my_y = lax.axis_index("y")
        xn = (1 - my_x, my_y)
        yn = (my_x, 1 - my_y)

        barrier_sem = pltpu.get_barrier_semaphore()
        for nbr in (xn, yn):
            pl.semaphore_signal(
                barrier_sem, inc=1, device_id=nbr,
                device_id_type=pltpu.DeviceIdType.MESH,
            )
        pl.semaphore_wait(barrier_sem, 2)

        def rdma_x(c):
            sl = pl.ds(c * CH, CH)
            return pltpu.make_async_remote_copy(
                src_ref=partial_ref.at[sl, :], dst_ref=commx_ref.at[sl, :],
                send_sem=sx_send.at[c], recv_sem=sx_recv.at[c],
                device_id=xn, device_id_type=pltpu.DeviceIdType.MESH,
            )

        def rdma_y(c):
            sl = pl.ds(c * CH, CH)
            return pltpu.make_async_remote_copy(
                src_ref=sendy_ref.at[sl, :], dst_ref=commy_ref.at[sl, :],
                send_sem=sy_send.at[c], recv_sem=sy_recv.at[c],
                device_id=yn, device_id_type=pltpu.DeviceIdType.MESH,
            )

        for c in range(C):
            rdma_x(c).start()

        for c in range(C):
            sl = pl.ds(c * CH, CH)
            rdma_x(c).wait_recv()
            done = partial_ref[sl, :] + commx_ref[sl, :]
            sendy_ref[sl, :] = done
            rdma_y(c).start()
            out_ref[pl.ds(my_y * TB + c * CH, CH), :] = done.astype(
                jnp.float32
            )

        for c in range(C):
            sl = pl.ds(c * CH, CH)
            rdma_y(c).wait_recv()
            out_ref[pl.ds((1 - my_y) * TB + c * CH, CH), :] = commy_ref[
                sl, :
            ].astype(jnp.float32)

        for c in range(C):
            rdma_x(c).wait_send()
            rdma_y(c).wait_send()

    return pl.pallas_call(
        body,
        out_shape=jax.ShapeDtypeStruct((T, D), jnp.float32),
        in_specs=[pl.BlockSpec(memory_space=pltpu.VMEM)],
        out_specs=pl.BlockSpec(memory_space=pltpu.VMEM),
        scratch_shapes=[
            pltpu.VMEM((TB, D), jnp.bfloat16),
            pltpu.VMEM((TB, D), jnp.bfloat16),
            pltpu.VMEM((TB, D), jnp.bfloat16),
            pltpu.SemaphoreType.DMA((C,)),
            pltpu.SemaphoreType.DMA((C,)),
            pltpu.SemaphoreType.DMA((C,)),
            pltpu.SemaphoreType.DMA((C,)),
        ],
        compiler_params=pltpu.CompilerParams(collective_id=0),
    )(partial)
